# Initial kernel scaffold; baseline (speedup 1.0000x reference)
import jax
import jax.numpy as jnp
from jax.experimental import pallas as pl


def kernel(xn, xe_attr, xe_src, xe_dst, fc1_w, fc1_b, dl_w1, dl_w2):
    raise NotImplementedError("write your pallas kernel here")


# speedup vs baseline: 3.1976x; 3.1976x over previous
import functools

import jax
import jax.numpy as jnp
from jax import lax
from jax.experimental import pallas as pl
from jax.experimental.pallas import tpu as pltpu
from jax.experimental.pallas import tpu_sc as plsc

_N = 10000
_E = 320000
_D = 128
_A = 33

_NC = 2
_NS = 16
_NW = _NC * _NS
_C = 640
_K = _C // 128
_CPW = 16
_EPW = _C * _CPW
_E_PAD = _NW * _EPW
_HALF = 5120
_ACC_R = 5248
_T = 512


def _sc_gather(xn, idx_src2, idx_dst2):
    mesh = plsc.VectorSubcoreMesh(core_axis_name="c", subcore_axis_name="s")

    @functools.partial(
        pl.kernel,
        out_type=[jax.ShapeDtypeStruct((_E_PAD, _D), jnp.float32),
                  jax.ShapeDtypeStruct((_E_PAD, _D), jnp.float32)],
        mesh=mesh,
        scratch_types=[pltpu.VMEM((8, 128), jnp.int32),
                       pltpu.VMEM((_C, _D), jnp.float32),
                       pltpu.SemaphoreType.DMA],
    )
    def k(xn_hbm, is_hbm, id_hbm, os_hbm, od_hbm, idx_v, rows_v, sem):
        w = lax.axis_index("s") * _NC + lax.axis_index("c")
        base = w * _EPW

        def run(i_hbm, o_hbm):
            def body(ck, carry):
                off = base + ck * _C
                pltpu.sync_copy(i_hbm.at[w * _CPW + ck], idx_v)
                descs = [
                    pltpu.async_copy(xn_hbm.at[idx_v.at[j]],
                                     rows_v.at[pl.ds(j * 128, 128)], sem)
                    for j in range(_K)
                ]
                for d in descs:
                    d.wait()
                pltpu.sync_copy(rows_v, o_hbm.at[pl.ds(off, _C)])
                return carry

            lax.fori_loop(0, _CPW, body, 0)

        run(is_hbm, os_hbm)
        run(id_hbm, od_hbm)

    return k(xn, idx_src2, idx_dst2)


def _sc_scatter(a_dst, a_src, idx_dst2, idx_src2, zrows):
    mesh = plsc.VectorSubcoreMesh(core_axis_name="c", subcore_axis_name="s")
    z_per_sub = _ACC_R // _NS
    o_per_sub = _HALF // _NS
    cpw = _E_PAD // (_NS * _C)
    epw = cpw * _C

    @functools.partial(
        pl.kernel,
        out_type=jax.ShapeDtypeStruct((2 * _HALF, _D), jnp.float32),
        mesh=mesh,
        scratch_types=[pltpu.VMEM((8, 128), jnp.int32),
                       pltpu.VMEM((_C, _D), jnp.float32),
                       pltpu.VMEM_SHARED((_ACC_R, _D), jnp.float32)],
    )
    def k(ad_hbm, as_hbm, id_hbm, is_hbm, z_hbm, out_hbm, idx_v, rows_v, acc):
        c = lax.axis_index("c")
        s = lax.axis_index("s")
        base = s * epw

        pltpu.sync_copy(z_hbm.at[pl.ds(s * z_per_sub, z_per_sub)],
                        acc.at[pl.ds(s * z_per_sub, z_per_sub)])
        plsc.subcore_barrier()

        def run(r_hbm, i_hbm):
            def body(ck, carry):
                off = base + ck * _C
                pltpu.sync_copy(i_hbm.at[(c * _NS + s) * cpw + ck], idx_v)
                pltpu.sync_copy(r_hbm.at[pl.ds(off, _C)], rows_v)
                for j in range(_K):
                    pltpu.sync_copy(rows_v.at[pl.ds(j * 128, 128)],
                                    acc.at[idx_v.at[j]], add=True)
                return carry

            lax.fori_loop(0, cpw, body, 0)

        run(ad_hbm, id_hbm)
        run(as_hbm, is_hbm)
        plsc.subcore_barrier()
        pltpu.sync_copy(acc.at[pl.ds(s * o_per_sub, o_per_sub)],
                        out_hbm.at[pl.ds(c * _HALF + s * o_per_sub,
                                         o_per_sub)])

    return k(a_dst, a_src, idx_dst2, idx_src2, zrows)


def _tc_edge_body(attr_ref, xs_ref, xd_ref, w1_ref, b1_ref, w2_ref,
                  adst_ref, asrc_ref):
    attr = attr_ref[...]
    W = jnp.dot(attr, w1_ref[...], preferred_element_type=jnp.float32)
    W = W + b1_ref[...]
    W = W * jax.nn.sigmoid(W)
    xs = xs_ref[...]
    xd = xd_ref[...]
    g = W * (xs - xd)
    a = W * (xs + xd) * 0.5
    dxe = jnp.concatenate([g, a, g * a, g * g, a * a], axis=1)
    x = jnp.tanh(dxe)
    x = jnp.dot(x, w2_ref[...], preferred_element_type=jnp.float32)
    x = x - jnp.mean(x, axis=1, keepdims=True)
    x = x * lax.rsqrt(jnp.sum(x * x, axis=1, keepdims=True) + 0.001)
    x = jnp.tanh(x)
    x = jnp.dot(x, w2_ref[...], preferred_element_type=jnp.float32)
    x = jnp.tanh(x)
    x0 = x[:, :_D]
    s = 0.5 * (x[:, _D:2 * _D] + x[:, 2 * _D:3 * _D]
               + x[:, 3 * _D:4 * _D] + x[:, 4 * _D:])
    adst_ref[...] = W * (s + x0)
    asrc_ref[...] = W * (s - x0)


def _tc_edges(xe_attr, xs, xd, fc1_wT, fc1_b2, dl_w1T):
    grid = _E_PAD // _T
    last_real = _E // _T - 1
    return pl.pallas_call(
        _tc_edge_body,
        grid=(grid,),
        in_specs=[
            pl.BlockSpec((_T, _A), lambda i: (jnp.minimum(i, last_real), 0)),
            pl.BlockSpec((_T, _D), lambda i: (i, 0)),
            pl.BlockSpec((_T, _D), lambda i: (i, 0)),
            pl.BlockSpec((_A, _D), lambda i: (0, 0)),
            pl.BlockSpec((1, _D), lambda i: (0, 0)),
            pl.BlockSpec((5 * _D, 5 * _D), lambda i: (0, 0)),
        ],
        out_specs=[
            pl.BlockSpec((_T, _D), lambda i: (i, 0)),
            pl.BlockSpec((_T, _D), lambda i: (i, 0)),
        ],
        out_shape=[jax.ShapeDtypeStruct((_E_PAD, _D), jnp.float32),
                   jax.ShapeDtypeStruct((_E_PAD, _D), jnp.float32)],
        compiler_params=pltpu.CompilerParams(
            dimension_semantics=("arbitrary",)),
    )(xe_attr, xs, xd, fc1_wT, fc1_b2, dl_w1T)


def _chunk_idx(idx_pad):
    a = idx_pad.reshape(-1, _K, 128)
    pad = [(0, 0)] * (a.ndim - 2) + [(0, 8 - _K), (0, 0)]
    return jnp.pad(a, pad)


def _local_idx(idx_pad):
    halves = []
    for core in range(_NC):
        loc = idx_pad - core * _HALF
        ok = (loc >= 0) & (loc < _HALF)
        halves.append(jnp.where(ok, loc, _HALF))
    return _chunk_idx(jnp.stack(halves).reshape(-1, _K, 128))


def kernel(xn, xe_attr, xe_src, xe_dst, fc1_w, fc1_b, dl_w1, dl_w2):
    npad = _E_PAD - _E
    src_i = xe_src.astype(jnp.int32)
    dst_i = xe_dst.astype(jnp.int32)
    gsrc = _chunk_idx(jnp.pad(src_i, (0, npad)))
    gdst = _chunk_idx(jnp.pad(dst_i, (0, npad)))
    ssrc = _local_idx(jnp.pad(src_i, (0, npad), constant_values=_N))
    sdst = _local_idx(jnp.pad(dst_i, (0, npad), constant_values=_N))

    xs, xd = _sc_gather(xn, gsrc, gdst)
    a_dst, a_src = _tc_edges(xe_attr, xs, xd, fc1_w.T, fc1_b[None, :],
                             dl_w1.T)
    zrows = jnp.zeros((_ACC_R, _D), jnp.float32)
    acc = _sc_scatter(a_dst, a_src, sdst, ssrc, zrows)
    return acc[:_N]
